# Initial kernel scaffold; baseline (speedup 1.0000x reference)
import jax
import jax.numpy as jnp
from jax.experimental import pallas as pl


def kernel(x, edge_index, batch, W1, b1, W2, b2, Wl1, bl1, Wl2, bl2, Wl3, bl3, Wl4, bl4, Wl5, bl5):
    raise NotImplementedError("write your pallas kernel here")


# speedup vs baseline: 16.3946x; 16.3946x over previous
import jax
import jax.numpy as jnp
from jax import lax
from jax.experimental import pallas as pl
from jax.experimental.pallas import tpu as pltpu
from jax.experimental.pallas import tpu_sc as plsc

_N = 10000
_E = 320000
_G = 64
_RP = 10240
_NC = 2
_NS = 16
_NW = _NC * _NS
_CH = 128
_EPAD = 327680
_CPT = _EPAD // _NS // _CH
_DUMMY = _RP - 8
_RSUB = _RP // _NS

_f32 = jnp.float32
_i32 = jnp.int32


def _deg_body(dst_hbm, deg_hbm, idx_v, deg_v):
    c = lax.axis_index("c")
    s = lax.axis_index("s")
    w = c * _NS + s
    pltpu.sync_copy(dst_hbm.at[w], idx_v)
    zero16 = jnp.zeros((16,), _f32)

    def zbody(i, carry):
        deg_v[pl.ds(i * 16, 16)] = zero16
        return carry

    lax.fori_loop(0, _RP // 16, zbody, 0)
    ones16 = jnp.ones((16,), _f32)

    def body(t, carry):
        idx16 = idx_v[t]
        plsc.addupdate_scatter(deg_v, [idx16], ones16)
        return carry

    lax.fori_loop(0, _EPAD // _NW // 16, body, 0)
    pltpu.sync_copy(deg_v, deg_hbm.at[w])


_SC_PARAMS = pltpu.CompilerParams(
    needs_layout_passes=False, use_tc_tiling_on_sc=False
)

_deg_call = pl.kernel(
    _deg_body,
    mesh=plsc.VectorSubcoreMesh(core_axis_name="c", subcore_axis_name="s"),
    out_type=jax.ShapeDtypeStruct((_NW, _RP), _f32),
    scratch_types=[
        pltpu.VMEM((_EPAD // _NW // 16, 16), _i32),
        pltpu.VMEM((_RP,), _f32),
    ],
    compiler_params=_SC_PARAMS,
)


def _make_scatter(d):
    dh = d // 2

    def body(g_hbm, ei_hbm, out_hbm, idx_v, buf0, buf1, sem0, sem1, acc):
        c = lax.axis_index("c")
        s = lax.axis_index("s")
        gtab = g_hbm.at[c]
        pltpu.sync_copy(ei_hbm.at[s], idx_v)

        zero16 = jnp.zeros((16,), _f32)
        nper = dh // 16

        def zbody(i, carry):
            buf0[i // nper, pl.ds((i % nper) * 16, 16)] = zero16
            return carry

        lax.fori_loop(0, _CH * nper, zbody, 0)
        base = s * _RSUB
        for r in range(_RSUB // _CH):
            pltpu.sync_copy(buf0, acc.at[pl.ds(base + r * _CH, _CH)])
        plsc.subcore_barrier()

        bufs = (buf0, buf1)
        sems = (sem0, sem1)
        for b in range(2):
            pltpu.make_async_copy(
                gtab.at[idx_v.at[b].at[0]], bufs[b], sems[b]).start()

        def loop_body(i, carry):
            j = i * 2
            for b in range(2):
                jj = j + b
                pltpu.make_async_copy(
                    gtab.at[idx_v.at[jj].at[0]], bufs[b], sems[b]).wait()
                pltpu.sync_copy(bufs[b], acc.at[idx_v.at[jj].at[1]], add=True)

                @pl.when(jj + 2 < _CPT)
                def _():
                    pltpu.make_async_copy(
                        gtab.at[idx_v.at[jj + 2].at[0]], bufs[b], sems[b]
                    ).start()

            return carry

        lax.fori_loop(0, _CPT // 2, loop_body, 0)
        plsc.subcore_barrier()
        for r in range(_RSUB // _CH):
            row = base + r * _CH
            pltpu.sync_copy(acc.at[pl.ds(row, _CH)], buf0)
            pltpu.sync_copy(buf0, out_hbm.at[c].at[pl.ds(row, _CH)])

    return pl.kernel(
        body,
        mesh=plsc.VectorSubcoreMesh(core_axis_name="c", subcore_axis_name="s"),
        out_type=jax.ShapeDtypeStruct((_NC, _RP, dh), _f32),
        scratch_types=[
            pltpu.VMEM((_CPT, 2, _CH), _i32),
            pltpu.VMEM((_CH, dh), _f32),
            pltpu.VMEM((_CH, dh), _f32),
            pltpu.SemaphoreType.DMA,
            pltpu.SemaphoreType.DMA,
            pltpu.VMEM_SHARED((_RP, dh), _f32),
        ],
        compiler_params=_SC_PARAMS,
    )


_scatter128 = _make_scatter(128)
_scatter64 = _make_scatter(64)


def _k1_body(degp_ref, xp_ref, w1_ref, g1_ref, dinv_ref):
    degp = degp_ref[...]
    ones = jnp.ones((_NW, 1), _f32)
    degc = lax.dot_general(degp, ones, (((0,), (0,)), ((), ())),
                           preferred_element_type=_f32)
    dinv = lax.rsqrt(degc + 1.0)
    h = jnp.dot(xp_ref[...], w1_ref[...], preferred_element_type=_f32)
    g1_ref[...] = h * dinv
    dinv_ref[...] = dinv


_k1 = pl.pallas_call(
    _k1_body,
    out_shape=[
        jax.ShapeDtypeStruct((_RP, 128), _f32),
        jax.ShapeDtypeStruct((_RP, 1), _f32),
    ],
)


def _k3_body(s1p_ref, g1_ref, dinv_ref, b1_ref, w2_ref, g2_ref):
    g1 = g1_ref[...]
    sacc = jnp.concatenate([s1p_ref[0], s1p_ref[1]], axis=1) + g1
    dinv = dinv_ref[...]
    a = jnp.maximum(dinv * sacc + b1_ref[...], 0.0)
    h2 = jnp.dot(a, w2_ref[...], preferred_element_type=_f32)
    g2_ref[...] = dinv * h2


_k3 = pl.pallas_call(
    _k3_body,
    out_shape=jax.ShapeDtypeStruct((_RP, 64), _f32),
)


def _k5_body(s2p_ref, g2_ref, dinv_ref, b2_ref, batch_ref,
             wl1_ref, bl1_ref, wl2_ref, bl2_ref, wl3_ref, bl3_ref,
             wl4_ref, bl4_ref, wl5_ref, bl5_ref, out_ref):
    sacc = jnp.concatenate([s2p_ref[0], s2p_ref[1]], axis=1) + g2_ref[...]
    o = dinv_ref[...] * sacc + b2_ref[...]
    bt = batch_ref[...]
    gid = lax.broadcasted_iota(_i32, (_G, _RP), 0)
    m = jnp.where(gid == bt, 1.0, 0.0)
    h = jnp.dot(m, o, preferred_element_type=_f32)
    h = jnp.maximum(jnp.dot(h, wl1_ref[...], preferred_element_type=_f32)
                    + bl1_ref[...], 0.0)
    h = jnp.maximum(jnp.dot(h, wl2_ref[...], preferred_element_type=_f32)
                    + bl2_ref[...], 0.0)
    h = jnp.maximum(jnp.dot(h, wl3_ref[...], preferred_element_type=_f32)
                    + bl3_ref[...], 0.0)
    h = jnp.maximum(jnp.dot(h, wl4_ref[...], preferred_element_type=_f32)
                    + bl4_ref[...], 0.0)
    out_ref[...] = (jnp.dot(h, wl5_ref[...], preferred_element_type=_f32)
                    + bl5_ref[...])


_k5 = pl.pallas_call(
    _k5_body,
    out_shape=jax.ShapeDtypeStruct((_G, 128), _f32),
)


def kernel(x, edge_index, batch, W1, b1, W2, b2, Wl1, bl1, Wl2, bl2,
           Wl3, bl3, Wl4, bl4, Wl5, bl5):
    src = edge_index[0].astype(_i32)
    dst = edge_index[1].astype(_i32)
    pad = _EPAD - _E
    fill = jnp.full((pad,), _DUMMY, _i32)
    src_p = jnp.concatenate([src, fill]).reshape(_NS, _CPT, _CH)
    dst_p = jnp.concatenate([dst, fill]).reshape(_NS, _CPT, _CH)
    ei = jnp.stack([src_p, dst_p], axis=2)
    dst_d = dst_p.reshape(_NW, _EPAD // _NW // 16, 16)

    xp = jnp.pad(x, ((0, _RP - _N), (0, 256 - 131)))
    w1p = jnp.pad(W1, ((0, 256 - 131), (0, 0)))
    batch_p = jnp.pad(batch.astype(_i32), (0, _RP - _N),
                      constant_values=_G).reshape(1, _RP)
    wl5p = jnp.pad(Wl5, ((0, 0), (0, 127)))
    bl5p = jnp.pad(bl5, (0, 127)).reshape(1, -1)

    deg_p = _deg_call(dst_d)
    g1, dinv = _k1(deg_p, xp, w1p)
    g1h = jnp.stack([g1[:, :64], g1[:, 64:]])
    s1p = _scatter128(g1h, ei)
    g2 = _k3(s1p, g1, dinv, b1.reshape(1, -1), W2)
    g2h = jnp.stack([g2[:, :32], g2[:, 32:]])
    s2p = _scatter64(g2h, ei)
    out = _k5(s2p, g2, dinv, b2.reshape(1, -1), batch_p,
              Wl1, bl1.reshape(1, -1), Wl2, bl2.reshape(1, -1),
              Wl3, bl3.reshape(1, -1), Wl4, bl4.reshape(1, -1),
              wl5p, bl5p)
    return out[:, :1]


# speedup vs baseline: 16.6725x; 1.0170x over previous
import jax
import jax.numpy as jnp
from jax import lax
from jax.experimental import pallas as pl
from jax.experimental.pallas import tpu as pltpu
from jax.experimental.pallas import tpu_sc as plsc

_N = 10000
_E = 320000
_G = 64
_RP = 10240
_NC = 2
_NS = 16
_NW = _NC * _NS
_CH = 128
_EPAD = 327680
_CPT = _EPAD // _NS // _CH
_DUMMY = _RP - 8
_RSUB = _RP // _NS

_f32 = jnp.float32
_i32 = jnp.int32


def _deg_body(dst_hbm, deg_hbm, idx_v, deg_v):
    c = lax.axis_index("c")
    s = lax.axis_index("s")
    w = c * _NS + s
    pltpu.sync_copy(dst_hbm.at[w], idx_v)
    zero16 = jnp.zeros((16,), _f32)

    def zbody(i, carry):
        deg_v[pl.ds(i * 16, 16)] = zero16
        return carry

    lax.fori_loop(0, _RP // 16, zbody, 0)
    ones16 = jnp.ones((16,), _f32)

    def body(t, carry):
        idx16 = idx_v[t]
        plsc.addupdate_scatter(deg_v, [idx16], ones16)
        return carry

    lax.fori_loop(0, _EPAD // _NW // 16, body, 0)
    pltpu.sync_copy(deg_v, deg_hbm.at[w])


_SC_PARAMS = pltpu.CompilerParams(
    needs_layout_passes=False, use_tc_tiling_on_sc=False
)

_deg_call = pl.kernel(
    _deg_body,
    mesh=plsc.VectorSubcoreMesh(core_axis_name="c", subcore_axis_name="s"),
    out_type=jax.ShapeDtypeStruct((_NW, _RP), _f32),
    scratch_types=[
        pltpu.VMEM((_EPAD // _NW // 16, 16), _i32),
        pltpu.VMEM((_RP,), _f32),
    ],
    compiler_params=_SC_PARAMS,
)


def _make_scatter(d):
    dh = d // 2

    def body(g_hbm, ei_hbm, out_hbm, idx_v, buf0, buf1, buf2, buf3,
             gs0, gs1, gs2, gs3, ss0, ss1, ss2, ss3, acc):
        c = lax.axis_index("c")
        s = lax.axis_index("s")
        gtab = g_hbm.at[c]
        pltpu.sync_copy(ei_hbm.at[s], idx_v)

        zero16 = jnp.zeros((16,), _f32)
        nper = dh // 16

        def zbody(i, carry):
            buf0[i // nper, pl.ds((i % nper) * 16, 16)] = zero16
            return carry

        lax.fori_loop(0, _CH * nper, zbody, 0)
        base = s * _RSUB
        for r in range(_RSUB // _CH):
            pltpu.sync_copy(buf0, acc.at[pl.ds(base + r * _CH, _CH)])
        plsc.subcore_barrier()

        bufs = (buf0, buf1, buf2, buf3)
        gsem = (gs0, gs1, gs2, gs3)
        ssem = (ss0, ss1, ss2, ss3)

        def gather(j, b):
            return pltpu.make_async_copy(
                gtab.at[idx_v.at[j].at[0]], bufs[b], gsem[b])

        def scat(j, b):
            return pltpu.make_async_copy(
                bufs[b], acc.at[idx_v.at[j].at[1]], ssem[b])

        gather(0, 0).start()
        gather(1, 1).start()

        def loop_body(i, carry):
            j0 = i * 4
            for u in range(4):
                j = j0 + u
                if u < 2:
                    @pl.when(j >= 2)
                    def _():
                        scat(j - 2, (u + 2) % 4).wait()
                else:
                    scat(j - 2, (u + 2) % 4).wait()

                @pl.when(j + 2 < _CPT)
                def _():
                    gather(j + 2, (u + 2) % 4).start()

                gather(j, u).wait()
                scat(j, u).start(add=True)
            return carry

        lax.fori_loop(0, _CPT // 4, loop_body, 0)
        scat(_CPT - 2, 2).wait()
        scat(_CPT - 1, 3).wait()
        plsc.subcore_barrier()
        for r in range(_RSUB // _CH):
            row = base + r * _CH
            pltpu.sync_copy(acc.at[pl.ds(row, _CH)], buf0)
            pltpu.sync_copy(buf0, out_hbm.at[c].at[pl.ds(row, _CH)])

    return pl.kernel(
        body,
        mesh=plsc.VectorSubcoreMesh(core_axis_name="c", subcore_axis_name="s"),
        out_type=jax.ShapeDtypeStruct((_NC, _RP, dh), _f32),
        scratch_types=[
            pltpu.VMEM((_CPT, 2, _CH), _i32),
            pltpu.VMEM((_CH, dh), _f32),
            pltpu.VMEM((_CH, dh), _f32),
            pltpu.VMEM((_CH, dh), _f32),
            pltpu.VMEM((_CH, dh), _f32),
            pltpu.SemaphoreType.DMA,
            pltpu.SemaphoreType.DMA,
            pltpu.SemaphoreType.DMA,
            pltpu.SemaphoreType.DMA,
            pltpu.SemaphoreType.DMA,
            pltpu.SemaphoreType.DMA,
            pltpu.SemaphoreType.DMA,
            pltpu.SemaphoreType.DMA,
            pltpu.VMEM_SHARED((_RP, dh), _f32),
        ],
        compiler_params=_SC_PARAMS,
    )


_scatter128 = _make_scatter(128)
_scatter64 = _make_scatter(64)


def _k1_body(degp_ref, xp_ref, w1_ref, g1_ref, dinv_ref):
    degp = degp_ref[...]
    ones = jnp.ones((_NW, 1), _f32)
    degc = lax.dot_general(degp, ones, (((0,), (0,)), ((), ())),
                           preferred_element_type=_f32)
    dinv = lax.rsqrt(degc + 1.0)
    h = jnp.dot(xp_ref[...], w1_ref[...], preferred_element_type=_f32)
    g = h * dinv
    g1_ref[0] = g[:, :64]
    g1_ref[1] = g[:, 64:]
    dinv_ref[...] = dinv


_k1 = pl.pallas_call(
    _k1_body,
    out_shape=[
        jax.ShapeDtypeStruct((2, _RP, 64), _f32),
        jax.ShapeDtypeStruct((_RP, 1), _f32),
    ],
)


def _k3_body(s1p_ref, g1_ref, dinv_ref, b1_ref, w2_ref, g2_ref):
    sacc = jnp.concatenate(
        [s1p_ref[0] + g1_ref[0], s1p_ref[1] + g1_ref[1]], axis=1)
    dinv = dinv_ref[...]
    a = jnp.maximum(dinv * sacc + b1_ref[...], 0.0)
    h2 = jnp.dot(a, w2_ref[...], preferred_element_type=_f32)
    g2 = dinv * h2
    g2_ref[0] = g2[:, :32]
    g2_ref[1] = g2[:, 32:]


_k3 = pl.pallas_call(
    _k3_body,
    out_shape=jax.ShapeDtypeStruct((2, _RP, 32), _f32),
)


def _k5_body(s2p_ref, g2_ref, dinv_ref, b2_ref, batch_ref,
             wl1_ref, bl1_ref, wl2_ref, bl2_ref, wl3_ref, bl3_ref,
             wl4_ref, bl4_ref, wl5_ref, bl5_ref, out_ref):
    sacc = jnp.concatenate(
        [s2p_ref[0] + g2_ref[0], s2p_ref[1] + g2_ref[1]], axis=1)
    o = dinv_ref[...] * sacc + b2_ref[...]
    bt = batch_ref[...]
    gid = lax.broadcasted_iota(_i32, (_G, _RP), 0)
    m = jnp.where(gid == bt, 1.0, 0.0)
    h = jnp.dot(m, o, preferred_element_type=_f32)
    h = jnp.maximum(jnp.dot(h, wl1_ref[...], preferred_element_type=_f32)
                    + bl1_ref[...], 0.0)
    h = jnp.maximum(jnp.dot(h, wl2_ref[...], preferred_element_type=_f32)
                    + bl2_ref[...], 0.0)
    h = jnp.maximum(jnp.dot(h, wl3_ref[...], preferred_element_type=_f32)
                    + bl3_ref[...], 0.0)
    h = jnp.maximum(jnp.dot(h, wl4_ref[...], preferred_element_type=_f32)
                    + bl4_ref[...], 0.0)
    out_ref[...] = (jnp.dot(h, wl5_ref[...], preferred_element_type=_f32)
                    + bl5_ref[...])


_k5 = pl.pallas_call(
    _k5_body,
    out_shape=jax.ShapeDtypeStruct((_G, 128), _f32),
)


def kernel(x, edge_index, batch, W1, b1, W2, b2, Wl1, bl1, Wl2, bl2,
           Wl3, bl3, Wl4, bl4, Wl5, bl5):
    src = edge_index[0].astype(_i32)
    dst = edge_index[1].astype(_i32)
    pad = _EPAD - _E
    fill = jnp.full((pad,), _DUMMY, _i32)
    src_p = jnp.concatenate([src, fill]).reshape(_NS, _CPT, _CH)
    dst_p = jnp.concatenate([dst, fill]).reshape(_NS, _CPT, _CH)
    ei = jnp.stack([src_p, dst_p], axis=2)
    dst_d = dst_p.reshape(_NW, _EPAD // _NW // 16, 16)

    xp = jnp.pad(x, ((0, _RP - _N), (0, 256 - 131)))
    w1p = jnp.pad(W1, ((0, 256 - 131), (0, 0)))
    batch_p = jnp.pad(batch.astype(_i32), (0, _RP - _N),
                      constant_values=_G).reshape(1, _RP)
    wl5p = jnp.pad(Wl5, ((0, 0), (0, 127)))
    bl5p = jnp.pad(bl5, (0, 127)).reshape(1, -1)

    deg_p = _deg_call(dst_d)
    g1h, dinv = _k1(deg_p, xp, w1p)
    s1p = _scatter128(g1h, ei)
    g2h = _k3(s1p, g1h, dinv, b1.reshape(1, -1), W2)
    s2p = _scatter64(g2h, ei)
    out = _k5(s2p, g2h, dinv, b2.reshape(1, -1), batch_p,
              Wl1, bl1.reshape(1, -1), Wl2, bl2.reshape(1, -1),
              Wl3, bl3.reshape(1, -1), Wl4, bl4.reshape(1, -1),
              wl5p, bl5p)
    return out[:, :1]


# speedup vs baseline: 18.2705x; 1.0958x over previous
import jax
import jax.numpy as jnp
from jax import lax
from jax.experimental import pallas as pl
from jax.experimental.pallas import tpu as pltpu
from jax.experimental.pallas import tpu_sc as plsc

_N = 10000
_E = 320000
_G = 64
_RP = 10240
_NC = 2
_NS = 16
_NW = _NC * _NS
_CH = 128
_CH2 = 256
_EPAD = 327680
_CPT2 = _EPAD // _NS // _CH2
_DUMMY = _RP - 8
_RSUB = _RP // _NS

_f32 = jnp.float32
_i32 = jnp.int32


def _deg_body(dst_hbm, deg_hbm, idx_v, deg_v):
    c = lax.axis_index("c")
    s = lax.axis_index("s")
    w = c * _NS + s
    pltpu.sync_copy(dst_hbm.at[w], idx_v)
    zero16 = jnp.zeros((16,), _f32)

    def zbody(i, carry):
        deg_v[pl.ds(i * 16, 16)] = zero16
        return carry

    lax.fori_loop(0, _RP // 16, zbody, 0)
    ones16 = jnp.ones((16,), _f32)

    def body(t, carry):
        idx16 = idx_v[t]
        plsc.addupdate_scatter(deg_v, [idx16], ones16)
        return carry

    lax.fori_loop(0, _EPAD // _NW // 16, body, 0)
    pltpu.sync_copy(deg_v, deg_hbm.at[w])


_SC_PARAMS = pltpu.CompilerParams(
    needs_layout_passes=False, use_tc_tiling_on_sc=False
)

_deg_call = pl.kernel(
    _deg_body,
    mesh=plsc.VectorSubcoreMesh(core_axis_name="c", subcore_axis_name="s"),
    out_type=jax.ShapeDtypeStruct((_NW, _RP), _f32),
    scratch_types=[
        pltpu.VMEM((_EPAD // _NW // 16, 16), _i32),
        pltpu.VMEM((_RP,), _f32),
    ],
    compiler_params=_SC_PARAMS,
)


def _make_scatter(d):
    dh = d // 2

    def body(g_hbm, eis_hbm, eid_hbm, out_hbm, idxs_v, idxd_v, buf0, buf1,
             sem0, sem1, acc):
        c = lax.axis_index("c")
        s = lax.axis_index("s")
        gtab = g_hbm.at[c]
        pltpu.sync_copy(eis_hbm.at[s], idxs_v)
        pltpu.sync_copy(eid_hbm.at[s], idxd_v)

        zero16 = jnp.zeros((16,), _f32)
        nper = dh // 16

        def zbody(i, carry):
            buf0[i // nper, pl.ds((i % nper) * 16, 16)] = zero16
            return carry

        lax.fori_loop(0, _CH * nper, zbody, 0)
        base = s * _RSUB
        for r in range(_RSUB // _CH):
            pltpu.sync_copy(buf0.at[pl.ds(0, _CH)],
                            acc.at[pl.ds(base + r * _CH, _CH)])
        plsc.subcore_barrier()

        bufs = (buf0, buf1)
        sems = (sem0, sem1)

        def gather(j, b):
            return pltpu.make_async_copy(
                gtab.at[idxs_v.at[j]], bufs[b], sems[b])

        gather(0, 0).start()
        gather(1, 1).start()

        def loop_body(i, carry):
            j = i * 2
            for b in range(2):
                jj = j + b
                gather(jj, b).wait()
                pltpu.sync_copy(bufs[b].at[pl.ds(0, 128)],
                                acc.at[idxd_v.at[2 * jj]], add=True)
                pltpu.sync_copy(bufs[b].at[pl.ds(128, 128)],
                                acc.at[idxd_v.at[2 * jj + 1]], add=True)

                @pl.when(jj + 2 < _CPT2)
                def _():
                    gather(jj + 2, b).start()

            return carry

        lax.fori_loop(0, _CPT2 // 2, loop_body, 0)
        plsc.subcore_barrier()
        for r in range(_RSUB // _CH):
            row = base + r * _CH
            pltpu.sync_copy(acc.at[pl.ds(row, _CH)], buf0.at[pl.ds(0, _CH)])
            pltpu.sync_copy(buf0.at[pl.ds(0, _CH)],
                            out_hbm.at[c].at[pl.ds(row, _CH)])

    return pl.kernel(
        body,
        mesh=plsc.VectorSubcoreMesh(core_axis_name="c", subcore_axis_name="s"),
        out_type=jax.ShapeDtypeStruct((_NC, _RP, dh), _f32),
        scratch_types=[
            pltpu.VMEM((_CPT2, 256), _i32),
            pltpu.VMEM((2 * _CPT2, 128), _i32),
            pltpu.VMEM((_CH2, dh), _f32),
            pltpu.VMEM((_CH2, dh), _f32),
            pltpu.SemaphoreType.DMA,
            pltpu.SemaphoreType.DMA,
            pltpu.VMEM_SHARED((_RP, dh), _f32),
        ],
        compiler_params=_SC_PARAMS,
    )


_scatter128 = _make_scatter(128)
_scatter64 = _make_scatter(64)


def _k1_body(degp_ref, xp_ref, w1_ref, g1_ref, dinv_ref):
    degp = degp_ref[...]
    ones = jnp.ones((_NW, 1), _f32)
    degc = lax.dot_general(degp, ones, (((0,), (0,)), ((), ())),
                           preferred_element_type=_f32)
    dinv = lax.rsqrt(degc + 1.0)
    h = jnp.dot(xp_ref[...], w1_ref[...], preferred_element_type=_f32)
    g = h * dinv
    g1_ref[0] = g[:, :64]
    g1_ref[1] = g[:, 64:]
    dinv_ref[...] = dinv


_k1 = pl.pallas_call(
    _k1_body,
    out_shape=[
        jax.ShapeDtypeStruct((2, _RP, 64), _f32),
        jax.ShapeDtypeStruct((_RP, 1), _f32),
    ],
)


def _k3_body(s1p_ref, g1_ref, dinv_ref, b1_ref, w2_ref, g2_ref):
    sacc = jnp.concatenate(
        [s1p_ref[0] + g1_ref[0], s1p_ref[1] + g1_ref[1]], axis=1)
    dinv = dinv_ref[...]
    a = jnp.maximum(dinv * sacc + b1_ref[...], 0.0)
    h2 = jnp.dot(a, w2_ref[...], preferred_element_type=_f32)
    g2 = dinv * h2
    g2_ref[0] = g2[:, :32]
    g2_ref[1] = g2[:, 32:]


_k3 = pl.pallas_call(
    _k3_body,
    out_shape=jax.ShapeDtypeStruct((2, _RP, 32), _f32),
)


def _k5_body(s2p_ref, g2_ref, dinv_ref, b2_ref, batch_ref,
             wl1_ref, bl1_ref, wl2_ref, bl2_ref, wl3_ref, bl3_ref,
             wl4_ref, bl4_ref, wl5_ref, bl5_ref, out_ref):
    sacc = jnp.concatenate(
        [s2p_ref[0] + g2_ref[0], s2p_ref[1] + g2_ref[1]], axis=1)
    o = dinv_ref[...] * sacc + b2_ref[...]
    bt = batch_ref[...]
    gid = lax.broadcasted_iota(_i32, (_G, _RP), 0)
    m = jnp.where(gid == bt, 1.0, 0.0)
    h = jnp.dot(m, o, preferred_element_type=_f32)
    h = jnp.maximum(jnp.dot(h, wl1_ref[...], preferred_element_type=_f32)
                    + bl1_ref[...], 0.0)
    h = jnp.maximum(jnp.dot(h, wl2_ref[...], preferred_element_type=_f32)
                    + bl2_ref[...], 0.0)
    h = jnp.maximum(jnp.dot(h, wl3_ref[...], preferred_element_type=_f32)
                    + bl3_ref[...], 0.0)
    h = jnp.maximum(jnp.dot(h, wl4_ref[...], preferred_element_type=_f32)
                    + bl4_ref[...], 0.0)
    out_ref[...] = (jnp.dot(h, wl5_ref[...], preferred_element_type=_f32)
                    + bl5_ref[...])


_k5 = pl.pallas_call(
    _k5_body,
    out_shape=jax.ShapeDtypeStruct((_G, 128), _f32),
)


def kernel(x, edge_index, batch, W1, b1, W2, b2, Wl1, bl1, Wl2, bl2,
           Wl3, bl3, Wl4, bl4, Wl5, bl5):
    src = edge_index[0].astype(_i32)
    dst = edge_index[1].astype(_i32)
    pad = _EPAD - _E
    fill = jnp.full((pad,), _DUMMY, _i32)
    eis = jnp.concatenate([src, fill]).reshape(_NS, _CPT2, 256)
    dst_f = jnp.concatenate([dst, fill])
    eid = dst_f.reshape(_NS, 2 * _CPT2, 128)
    dst_d = dst_f.reshape(_NW, _EPAD // _NW // 16, 16)

    xp = jnp.pad(x, ((0, _RP - _N), (0, 256 - 131)))
    w1p = jnp.pad(W1, ((0, 256 - 131), (0, 0)))
    batch_p = jnp.pad(batch.astype(_i32), (0, _RP - _N),
                      constant_values=_G).reshape(1, _RP)
    wl5p = jnp.pad(Wl5, ((0, 0), (0, 127)))
    bl5p = jnp.pad(bl5, (0, 127)).reshape(1, -1)

    deg_p = _deg_call(dst_d)
    g1h, dinv = _k1(deg_p, xp, w1p)
    s1p = _scatter128(g1h, eis, eid)
    g2h = _k3(s1p, g1h, dinv, b1.reshape(1, -1), W2)
    s2p = _scatter64(g2h, eis, eid)
    out = _k5(s2p, g2h, dinv, b2.reshape(1, -1), batch_p,
              Wl1, bl1.reshape(1, -1), Wl2, bl2.reshape(1, -1),
              Wl3, bl3.reshape(1, -1), Wl4, bl4.reshape(1, -1),
              wl5p, bl5p)
    return out[:, :1]


# speedup vs baseline: 18.3509x; 1.0044x over previous
import jax
import jax.numpy as jnp
from jax import lax
from jax.experimental import pallas as pl
from jax.experimental.pallas import tpu as pltpu
from jax.experimental.pallas import tpu_sc as plsc

_N = 10000
_E = 320000
_G = 64
_RP = 10240
_NC = 2
_NS = 16
_NW = _NC * _NS
_CH = 128
_CH2 = 256
_EPAD = 327680
_CPT2 = _EPAD // _NS // _CH2
_DUMMY = _RP - 8
_RSUB = _RP // _NS

_f32 = jnp.float32
_i32 = jnp.int32


def _deg_body(dst_hbm, deg_hbm, idx_v, deg_v):
    c = lax.axis_index("c")
    s = lax.axis_index("s")
    w = c * _NS + s
    pltpu.sync_copy(dst_hbm.at[w], idx_v)
    zero16 = jnp.zeros((16,), _f32)

    def zbody(i, carry):
        deg_v[pl.ds(i * 16, 16)] = zero16
        return carry

    lax.fori_loop(0, _RP // 16, zbody, 0)
    ones16 = jnp.ones((16,), _f32)

    def body(t, carry):
        idx16 = idx_v[t]
        plsc.addupdate_scatter(deg_v, [idx16], ones16)
        return carry

    lax.fori_loop(0, _EPAD // _NW // 16, body, 0)
    pltpu.sync_copy(deg_v, deg_hbm.at[w])


_SC_PARAMS = pltpu.CompilerParams(
    needs_layout_passes=False, use_tc_tiling_on_sc=False
)

_deg_call = pl.kernel(
    _deg_body,
    mesh=plsc.VectorSubcoreMesh(core_axis_name="c", subcore_axis_name="s"),
    out_type=jax.ShapeDtypeStruct((_NW, _RP), _f32),
    scratch_types=[
        pltpu.VMEM((_EPAD // _NW // 16, 16), _i32),
        pltpu.VMEM((_RP,), _f32),
    ],
    compiler_params=_SC_PARAMS,
)


def _make_scatter(d):
    dh = d // 2

    def body(g_hbm, eis_hbm, eid_hbm, out_hbm, idxs_v, idxd_v, buf0, buf1,
             sem0, sem1, acc):
        c = lax.axis_index("c")
        s = lax.axis_index("s")
        gtab = g_hbm.at[c]
        pltpu.sync_copy(eis_hbm.at[s], idxs_v)
        pltpu.sync_copy(eid_hbm.at[s], idxd_v)

        zero16 = jnp.zeros((16,), _f32)
        nper = dh // 16

        def zbody(i, carry):
            buf0[i // nper, pl.ds((i % nper) * 16, 16)] = zero16
            return carry

        lax.fori_loop(0, _CH * nper, zbody, 0)
        base = s * _RSUB
        for r in range(_RSUB // _CH):
            pltpu.sync_copy(buf0.at[pl.ds(0, _CH)],
                            acc.at[pl.ds(base + r * _CH, _CH)])
        plsc.subcore_barrier()

        bufs = (buf0, buf1)
        sems = (sem0, sem1)

        def gather(j, b):
            return pltpu.make_async_copy(
                gtab.at[idxs_v.at[j]], bufs[b], sems[b])

        gather(0, 0).start()
        gather(1, 1).start()

        def loop_body(i, carry):
            j = i * 2
            for b in range(2):
                jj = j + b
                gather(jj, b).wait()
                pltpu.sync_copy(bufs[b], acc.at[idxd_v.at[jj]], add=True)

                @pl.when(jj + 2 < _CPT2)
                def _():
                    gather(jj + 2, b).start()

            return carry

        lax.fori_loop(0, _CPT2 // 2, loop_body, 0)
        plsc.subcore_barrier()
        for r in range(_RSUB // _CH):
            row = base + r * _CH
            pltpu.sync_copy(acc.at[pl.ds(row, _CH)], buf0.at[pl.ds(0, _CH)])
            pltpu.sync_copy(buf0.at[pl.ds(0, _CH)],
                            out_hbm.at[c].at[pl.ds(row, _CH)])

    return pl.kernel(
        body,
        mesh=plsc.VectorSubcoreMesh(core_axis_name="c", subcore_axis_name="s"),
        out_type=jax.ShapeDtypeStruct((_NC, _RP, dh), _f32),
        scratch_types=[
            pltpu.VMEM((_CPT2, 256), _i32),
            pltpu.VMEM((_CPT2, 256), _i32),
            pltpu.VMEM((_CH2, dh), _f32),
            pltpu.VMEM((_CH2, dh), _f32),
            pltpu.SemaphoreType.DMA,
            pltpu.SemaphoreType.DMA,
            pltpu.VMEM_SHARED((_RP, dh), _f32),
        ],
        compiler_params=_SC_PARAMS,
    )


_scatter128 = _make_scatter(128)
_scatter64 = _make_scatter(64)


def _k1_body(degp_ref, xp_ref, w1_ref, g1_ref, dinv_ref):
    degp = degp_ref[...]
    ones = jnp.ones((_NW, 1), _f32)
    degc = lax.dot_general(degp, ones, (((0,), (0,)), ((), ())),
                           preferred_element_type=_f32)
    dinv = lax.rsqrt(degc + 1.0)
    h = jnp.dot(xp_ref[...], w1_ref[...], preferred_element_type=_f32)
    g = h * dinv
    g1_ref[0] = g[:, :64]
    g1_ref[1] = g[:, 64:]
    dinv_ref[...] = dinv


_k1 = pl.pallas_call(
    _k1_body,
    out_shape=[
        jax.ShapeDtypeStruct((2, _RP, 64), _f32),
        jax.ShapeDtypeStruct((_RP, 1), _f32),
    ],
)


def _k3_body(s1p_ref, g1_ref, dinv_ref, b1_ref, w2_ref, g2_ref):
    sacc = jnp.concatenate(
        [s1p_ref[0] + g1_ref[0], s1p_ref[1] + g1_ref[1]], axis=1)
    dinv = dinv_ref[...]
    a = jnp.maximum(dinv * sacc + b1_ref[...], 0.0)
    h2 = jnp.dot(a, w2_ref[...], preferred_element_type=_f32)
    g2 = dinv * h2
    g2_ref[0] = g2[:, :32]
    g2_ref[1] = g2[:, 32:]


_k3 = pl.pallas_call(
    _k3_body,
    out_shape=jax.ShapeDtypeStruct((2, _RP, 32), _f32),
)


def _k5_body(s2p_ref, g2_ref, dinv_ref, b2_ref, batch_ref,
             wl1_ref, bl1_ref, wl2_ref, bl2_ref, wl3_ref, bl3_ref,
             wl4_ref, bl4_ref, wl5_ref, bl5_ref, out_ref):
    sacc = jnp.concatenate(
        [s2p_ref[0] + g2_ref[0], s2p_ref[1] + g2_ref[1]], axis=1)
    o = dinv_ref[...] * sacc + b2_ref[...]
    bt = batch_ref[...]
    gid = lax.broadcasted_iota(_i32, (_G, _RP), 0)
    m = jnp.where(gid == bt, 1.0, 0.0)
    h = jnp.dot(m, o, preferred_element_type=_f32)
    h = jnp.maximum(jnp.dot(h, wl1_ref[...], preferred_element_type=_f32)
                    + bl1_ref[...], 0.0)
    h = jnp.maximum(jnp.dot(h, wl2_ref[...], preferred_element_type=_f32)
                    + bl2_ref[...], 0.0)
    h = jnp.maximum(jnp.dot(h, wl3_ref[...], preferred_element_type=_f32)
                    + bl3_ref[...], 0.0)
    h = jnp.maximum(jnp.dot(h, wl4_ref[...], preferred_element_type=_f32)
                    + bl4_ref[...], 0.0)
    out_ref[...] = (jnp.dot(h, wl5_ref[...], preferred_element_type=_f32)
                    + bl5_ref[...])


_k5 = pl.pallas_call(
    _k5_body,
    out_shape=jax.ShapeDtypeStruct((_G, 128), _f32),
)


def kernel(x, edge_index, batch, W1, b1, W2, b2, Wl1, bl1, Wl2, bl2,
           Wl3, bl3, Wl4, bl4, Wl5, bl5):
    src = edge_index[0].astype(_i32)
    dst = edge_index[1].astype(_i32)
    pad = _EPAD - _E
    fill = jnp.full((pad,), _DUMMY, _i32)
    eis = jnp.concatenate([src, fill]).reshape(_NS, _CPT2, 256)
    dst_f = jnp.concatenate([dst, fill])
    eid = dst_f.reshape(_NS, _CPT2, 256)
    dst_d = dst_f.reshape(_NW, _EPAD // _NW // 16, 16)

    xp = jnp.pad(x, ((0, _RP - _N), (0, 256 - 131)))
    w1p = jnp.pad(W1, ((0, 256 - 131), (0, 0)))
    batch_p = jnp.pad(batch.astype(_i32), (0, _RP - _N),
                      constant_values=_G).reshape(1, _RP)
    wl5p = jnp.pad(Wl5, ((0, 0), (0, 127)))
    bl5p = jnp.pad(bl5, (0, 127)).reshape(1, -1)

    deg_p = _deg_call(dst_d)
    g1h, dinv = _k1(deg_p, xp, w1p)
    s1p = _scatter128(g1h, eis, eid)
    g2h = _k3(s1p, g1h, dinv, b1.reshape(1, -1), W2)
    s2p = _scatter64(g2h, eis, eid)
    out = _k5(s2p, g2h, dinv, b2.reshape(1, -1), batch_p,
              Wl1, bl1.reshape(1, -1), Wl2, bl2.reshape(1, -1),
              Wl3, bl3.reshape(1, -1), Wl4, bl4.reshape(1, -1),
              wl5p, bl5p)
    return out[:, :1]
